# baseline (speedup 1.0000x reference)
import functools

import jax
import jax.numpy as jnp
from jax import lax
from jax.experimental import pallas as pl
from jax.experimental.pallas import tpu as pltpu
from jax.experimental.pallas import tpu_sc as plsc

N = 256 * 24 * 325
NC, NS, L = 2, 16, 16
NW = NC * NS
PER_W = N // NW
CHUNK = 6240
NCHUNK = PER_W // CHUNK
VECS = CHUNK // L
PROW = 2 * L


def _lane_shuffle(x, idx):
    dnums = lax.GatherDimensionNumbers(
        offset_dims=(), collapsed_slice_dims=(0,), start_index_map=(0,))
    return lax.gather(x, idx[:, None], dimension_numbers=dnums,
                      slice_sizes=(1,),
                      mode=lax.GatherScatterMode.PROMISE_IN_BOUNDS)


def _lane_sum_all(x):
    iota = lax.iota(jnp.int32, L)
    for shift in (8, 4, 2, 1):
        x = x + _lane_shuffle(x, iota ^ shift)
    return x


def _mae_body(pred_hbm, true_hbm, out_hbm,
              pred0, pred1, true0, true1, partial_v, red_v, out_v,
              shared, sems):
    wid = lax.axis_index("s") * NC + lax.axis_index("c")
    base = wid * PER_W
    pred_bufs = (pred0, pred1)
    true_bufs = (true0, true1)

    def start(slot, j):
        off = base + j * CHUNK
        pltpu.make_async_copy(
            pred_hbm.at[pl.ds(off, CHUNK)], pred_bufs[slot],
            sems.at[slot, 0]).start()
        pltpu.make_async_copy(
            true_hbm.at[pl.ds(off, CHUNK)], true_bufs[slot],
            sems.at[slot, 1]).start()

    def wait(slot):
        pltpu.make_async_copy(
            pred_hbm.at[pl.ds(0, CHUNK)], pred_bufs[slot],
            sems.at[slot, 0]).wait()
        pltpu.make_async_copy(
            true_hbm.at[pl.ds(0, CHUNK)], true_bufs[slot],
            sems.at[slot, 1]).wait()

    start(0, 0)
    zero = jnp.zeros((L,), jnp.float32)
    acc, cnt = zero, zero
    for j in range(NCHUNK):
        slot = j % 2
        if j + 1 < NCHUNK:
            start(1 - slot, j + 1)
        wait(slot)
        pv, tv = pred_bufs[slot], true_bufs[slot]

        def vec_step(i, c, pv=pv, tv=tv):
            a, k = c
            p = pv[pl.ds(i * L, L)]
            t = tv[pl.ds(i * L, L)]
            m = t != 0.0
            a = a + jnp.where(m, jnp.abs(p - t), 0.0)
            k = k + jnp.where(m, 1.0, 0.0)
            return a, k

        acc, cnt = lax.fori_loop(0, VECS, vec_step, (acc, cnt))

    partial_v[pl.ds(0, L)] = acc
    partial_v[pl.ds(L, L)] = cnt
    pltpu.sync_copy(partial_v, shared.at[pl.ds(wid * PROW, PROW)])
    plsc.subcore_barrier()

    @pl.when(wid == 0)
    def _():
        pltpu.sync_copy(shared, red_v)
        acc0, cnt0 = zero, zero
        for i in range(NW):
            acc0 = acc0 + red_v[pl.ds(i * PROW, L)]
            cnt0 = cnt0 + red_v[pl.ds(i * PROW + L, L)]
        out_v[...] = _lane_sum_all(acc0) / _lane_sum_all(cnt0)
        pltpu.sync_copy(out_v, out_hbm)


@jax.jit
def _mae_sc(pred_flat, true_flat):
    mesh = plsc.VectorSubcoreMesh(core_axis_name="c", subcore_axis_name="s")
    run = pl.kernel(
        _mae_body,
        out_type=jax.ShapeDtypeStruct((L,), jnp.float32),
        mesh=mesh,
        scratch_types=[
            pltpu.VMEM((CHUNK,), jnp.float32),
            pltpu.VMEM((CHUNK,), jnp.float32),
            pltpu.VMEM((CHUNK,), jnp.float32),
            pltpu.VMEM((CHUNK,), jnp.float32),
            pltpu.VMEM((PROW,), jnp.float32),
            pltpu.VMEM((NW * PROW,), jnp.float32),
            pltpu.VMEM((L,), jnp.float32),
            pltpu.VMEM_SHARED((NW * PROW,), jnp.float32),
            pltpu.SemaphoreType.DMA((2, 2)),
        ],
    )
    return run(pred_flat, true_flat)


def kernel(y_pred, y_true):
    out = _mae_sc(y_pred.reshape(N), y_true.reshape(N))
    return out[0]


# speedup vs baseline: 1.0678x; 1.0678x over previous
import functools

import jax
import jax.numpy as jnp
from jax import lax
from jax.experimental import pallas as pl
from jax.experimental.pallas import tpu as pltpu
from jax.experimental.pallas import tpu_sc as plsc

N = 256 * 24 * 325
NC, NS, L = 2, 16, 16
NW = NC * NS
PER_W = N // NW
CHUNK = 6240
NCHUNK = PER_W // CHUNK
VECS = CHUNK // L
UNROLL = 10
NACC = 4
PROW = 2 * L


def _lane_shuffle(x, idx):
    dnums = lax.GatherDimensionNumbers(
        offset_dims=(), collapsed_slice_dims=(0,), start_index_map=(0,))
    return lax.gather(x, idx[:, None], dimension_numbers=dnums,
                      slice_sizes=(1,),
                      mode=lax.GatherScatterMode.PROMISE_IN_BOUNDS)


def _lane_sum_all(x):
    iota = lax.iota(jnp.int32, L)
    for shift in (8, 4, 2, 1):
        x = x + _lane_shuffle(x, iota ^ shift)
    return x


def _mae_body(pred_hbm, true_hbm, out_hbm,
              pred0, pred1, true0, true1, partial_v, red_v, out_v,
              shared, sems):
    wid = lax.axis_index("s") * NC + lax.axis_index("c")
    base = wid * PER_W
    pred_bufs = (pred0, pred1)
    true_bufs = (true0, true1)

    def start(slot, j):
        off = base + j * CHUNK
        pltpu.make_async_copy(
            pred_hbm.at[pl.ds(off, CHUNK)], pred_bufs[slot],
            sems.at[slot, 0]).start()
        pltpu.make_async_copy(
            true_hbm.at[pl.ds(off, CHUNK)], true_bufs[slot],
            sems.at[slot, 1]).start()

    def wait(slot):
        pltpu.make_async_copy(
            pred_hbm.at[pl.ds(0, CHUNK)], pred_bufs[slot],
            sems.at[slot, 0]).wait()
        pltpu.make_async_copy(
            true_hbm.at[pl.ds(0, CHUNK)], true_bufs[slot],
            sems.at[slot, 1]).wait()

    start(0, 0)
    zero = jnp.zeros((L,), jnp.float32)
    accs = (zero,) * NACC + (zero,) * NACC
    for j in range(NCHUNK):
        slot = j % 2
        if j + 1 < NCHUNK:
            start(1 - slot, j + 1)
        wait(slot)
        pv, tv = pred_bufs[slot], true_bufs[slot]

        def vec_step(i, c, pv=pv, tv=tv):
            regs = list(c)
            for u in range(UNROLL):
                r = u % NACC
                p = pv[pl.ds((i + u) * L, L)]
                t = tv[pl.ds((i + u) * L, L)]
                m = t != 0.0
                regs[r] = regs[r] + jnp.where(m, jnp.abs(p - t), 0.0)
                regs[NACC + r] = regs[NACC + r] + jnp.where(m, 1.0, 0.0)
            return tuple(regs)

        accs = plsc.parallel_loop(0, VECS, step=UNROLL, carry=accs)(vec_step)

    acc = accs[0]
    cnt = accs[NACC]
    for r in range(1, NACC):
        acc = acc + accs[r]
        cnt = cnt + accs[NACC + r]

    partial_v[pl.ds(0, L)] = acc
    partial_v[pl.ds(L, L)] = cnt
    pltpu.sync_copy(partial_v, shared.at[pl.ds(wid * PROW, PROW)])
    plsc.subcore_barrier()

    @pl.when(wid == 0)
    def _():
        pltpu.sync_copy(shared, red_v)
        acc0, cnt0 = zero, zero
        for i in range(NW):
            acc0 = acc0 + red_v[pl.ds(i * PROW, L)]
            cnt0 = cnt0 + red_v[pl.ds(i * PROW + L, L)]
        out_v[...] = _lane_sum_all(acc0) / _lane_sum_all(cnt0)
        pltpu.sync_copy(out_v, out_hbm)


@jax.jit
def _mae_sc(pred_flat, true_flat):
    mesh = plsc.VectorSubcoreMesh(core_axis_name="c", subcore_axis_name="s")
    run = pl.kernel(
        _mae_body,
        out_type=jax.ShapeDtypeStruct((L,), jnp.float32),
        mesh=mesh,
        scratch_types=[
            pltpu.VMEM((CHUNK,), jnp.float32),
            pltpu.VMEM((CHUNK,), jnp.float32),
            pltpu.VMEM((CHUNK,), jnp.float32),
            pltpu.VMEM((CHUNK,), jnp.float32),
            pltpu.VMEM((PROW,), jnp.float32),
            pltpu.VMEM((NW * PROW,), jnp.float32),
            pltpu.VMEM((L,), jnp.float32),
            pltpu.VMEM_SHARED((NW * PROW,), jnp.float32),
            pltpu.SemaphoreType.DMA((2, 2)),
        ],
    )
    return run(pred_flat, true_flat)


def kernel(y_pred, y_true):
    out = _mae_sc(y_pred.reshape(N), y_true.reshape(N))
    return out[0]


# speedup vs baseline: 2.2952x; 2.1495x over previous
import functools

import jax
import jax.numpy as jnp
from jax import lax
from jax.experimental import pallas as pl
from jax.experimental.pallas import tpu as pltpu
from jax.experimental.pallas import tpu_sc as plsc

N = 256 * 24 * 325
NC, NS, L = 2, 16, 16
NW = NC * NS
PER_W = N // NW
CHUNK = 6240
NCHUNK = PER_W // CHUNK
VECS = CHUNK // L
UNROLL = 10
NACC = 4
PROW = 2 * L


def _lane_shuffle(x, idx):
    dnums = lax.GatherDimensionNumbers(
        offset_dims=(), collapsed_slice_dims=(0,), start_index_map=(0,))
    return lax.gather(x, idx[:, None], dimension_numbers=dnums,
                      slice_sizes=(1,),
                      mode=lax.GatherScatterMode.PROMISE_IN_BOUNDS)


def _lane_sum_all(x):
    iota = lax.iota(jnp.int32, L)
    for shift in (8, 4, 2, 1):
        x = x + _lane_shuffle(x, iota ^ shift)
    return x


def _mae_body(pred_hbm, true_hbm, out_hbm,
              pred0, pred1, true0, true1, partial_v, red_v, out_v,
              shared, sems):
    wid = lax.axis_index("s") * NC + lax.axis_index("c")
    base = wid * PER_W
    pred_bufs = (pred0, pred1)
    true_bufs = (true0, true1)

    def start(slot, j):
        off = base + j * CHUNK
        pltpu.make_async_copy(
            pred_hbm.at[pl.ds(off, CHUNK)], pred_bufs[slot],
            sems.at[slot, 0]).start()
        pltpu.make_async_copy(
            true_hbm.at[pl.ds(off, CHUNK)], true_bufs[slot],
            sems.at[slot, 1]).start()

    def wait(slot):
        pltpu.make_async_copy(
            pred_hbm.at[pl.ds(0, CHUNK)], pred_bufs[slot],
            sems.at[slot, 0]).wait()
        pltpu.make_async_copy(
            true_hbm.at[pl.ds(0, CHUNK)], true_bufs[slot],
            sems.at[slot, 1]).wait()

    start(0, 0)
    zero = jnp.zeros((L,), jnp.float32)
    accs = (zero,) * NACC + (zero,) * NACC
    for j in range(NCHUNK):
        slot = j % 2
        if j + 1 < NCHUNK:
            start(1 - slot, j + 1)
        wait(slot)
        pv, tv = pred_bufs[slot], true_bufs[slot]

        def vec_step(i, c, pv=pv, tv=tv):
            regs = list(c)
            for u in range(UNROLL):
                r = u % NACC
                p = pv[pl.ds((i + u) * L, L)]
                t = tv[pl.ds((i + u) * L, L)]
                m = t != 0.0
                regs[r] = regs[r] + jnp.where(m, jnp.abs(p - t), 0.0)
                regs[NACC + r] = regs[NACC + r] + jnp.where(m, 1.0, 0.0)
            return tuple(regs)

        accs = plsc.parallel_loop(0, VECS, step=UNROLL, carry=accs)(vec_step)

    acc = accs[0]
    cnt = accs[NACC]
    for r in range(1, NACC):
        acc = acc + accs[r]
        cnt = cnt + accs[NACC + r]

    partial_v[pl.ds(0, L)] = acc
    partial_v[pl.ds(L, L)] = cnt
    pltpu.sync_copy(partial_v, shared.at[pl.ds(wid * PROW, PROW)])
    plsc.subcore_barrier()

    @pl.when(wid == 0)
    def _():
        pltpu.sync_copy(shared, red_v)
        acc0, cnt0 = zero, zero
        for i in range(NW):
            acc0 = acc0 + red_v[pl.ds(i * PROW, L)]
            cnt0 = cnt0 + red_v[pl.ds(i * PROW + L, L)]
        out_v[...] = _lane_sum_all(acc0) / _lane_sum_all(cnt0)
        pltpu.sync_copy(out_v, out_hbm)


@jax.jit
def _mae_sc(pred_flat, true_flat):
    mesh = plsc.VectorSubcoreMesh(core_axis_name="c", subcore_axis_name="s")
    run = pl.kernel(
        _mae_body,
        out_type=jax.ShapeDtypeStruct((L,), jnp.float32),
        mesh=mesh,
        scratch_types=[
            pltpu.VMEM((CHUNK,), jnp.float32),
            pltpu.VMEM((CHUNK,), jnp.float32),
            pltpu.VMEM((CHUNK,), jnp.float32),
            pltpu.VMEM((CHUNK,), jnp.float32),
            pltpu.VMEM((PROW,), jnp.float32),
            pltpu.VMEM((NW * PROW,), jnp.float32),
            pltpu.VMEM((L,), jnp.float32),
            pltpu.VMEM_SHARED((NW * PROW,), jnp.float32),
            pltpu.SemaphoreType.DMA((2, 2)),
        ],
    )
    return run(pred_flat, true_flat)


def kernel(y_pred, y_true):
    p = y_pred.transpose(1, 2, 3, 0).reshape(N)
    t = y_true.transpose(1, 2, 3, 0).reshape(N)
    out = _mae_sc(p, t)
    return out[0]
